# baseline (speedup 1.0000x reference)
import functools

import jax
import jax.numpy as jnp
from jax import lax
from jax.experimental import pallas as pl
from jax.experimental.pallas import tpu as pltpu
from jax.experimental.pallas import tpu_sc as plsc

NUM_FIELDS = 26
VOCAB = 100000
EMBED_DIM = 32
BATCH = 4096

_NC = 2
_NS = 16
_NW = _NC * _NS
_BPW = BATCH // _NW
_ROWS = _BPW * NUM_FIELDS


def _gather_body(gidx_hbm, table_hbm, out_hbm, idxv, rows, sem):
    wid = lax.axis_index("s") * _NC + lax.axis_index("c")
    pltpu.sync_copy(gidx_hbm.at[wid], idxv)

    def step(j, carry):
        pltpu.async_copy(
            table_hbm.at[idxv.at[j]],
            rows.at[pl.ds(j * _BPW, _BPW)],
            sem,
        ).wait()
        return carry

    lax.fori_loop(0, NUM_FIELDS, step, 0)
    pltpu.sync_copy(rows, out_hbm.at[pl.ds(wid * _ROWS, _ROWS)])


_gather = pl.kernel(
    _gather_body,
    out_type=jax.ShapeDtypeStruct((BATCH * NUM_FIELDS, EMBED_DIM), jnp.float32),
    mesh=plsc.VectorSubcoreMesh(core_axis_name="c", subcore_axis_name="s"),
    scratch_types=[
        pltpu.VMEM((NUM_FIELDS, _BPW), jnp.int32),
        pltpu.VMEM((_ROWS, EMBED_DIM), jnp.float32),
        pltpu.SemaphoreType.DMA,
    ],
    compiler_params=pltpu.CompilerParams(use_tc_tiling_on_sc=False),
)


@jax.jit
def kernel(x, tables):
    x = x.astype(jnp.int32)
    offs = (jnp.arange(NUM_FIELDS, dtype=jnp.int32) * VOCAB)[:, None]
    gidx = (x + offs).T.reshape(_NW, NUM_FIELDS, _BPW)
    table2 = tables.reshape(NUM_FIELDS * VOCAB, EMBED_DIM)
    out = _gather(gidx, table2)
    return out.reshape(BATCH, NUM_FIELDS * EMBED_DIM)


# speedup vs baseline: 4.8632x; 4.8632x over previous
import jax
import jax.numpy as jnp
from jax import lax
from jax.experimental import pallas as pl
from jax.experimental.pallas import tpu as pltpu
from jax.experimental.pallas import tpu_sc as plsc

NUM_FIELDS = 26
VOCAB = 100000
EMBED_DIM = 32
BATCH = 4096

_NC = 2
_NS = 16
_NW = _NC * _NS
_NROWS = NUM_FIELDS * EMBED_DIM
_NTASK = _NROWS // 8
_CW = 2048
_NFULL = 48
_NCH = _NFULL + 1
_LASTW = 99968 - _NFULL * _CW
_TAIL0 = 99968


def _body(x_hbm, tab_hbm, tail_hbm, out_hbm,
          xv, buf0, buf1, stag0, stag1, pv, pb, hist, cur, tailv, cs,
          sem0, sem1, osem0, osem1):
    wid = lax.axis_index("s") * _NC + lax.axis_index("c")
    lane = lax.iota(jnp.int32, 16)
    ones = jnp.ones((16,), jnp.int32)
    zeros = jnp.zeros((16,), jnp.int32)
    stags = (stag0, stag1)
    osems = (osem0, osem1)

    def splat(s):
        return jnp.broadcast_to(jnp.int32(s) if isinstance(s, int) else s, (16,))

    def out_dma(ti, stag, osem):
        t = wid + _NW * ti
        return pltpu.make_async_copy(
            stag, out_hbm.at[pl.ds(t * 8, 8), :], osem
        )

    for ti in range(4):
        t = wid + _NW * ti
        stag = stags[ti % 2]
        osem = osems[ti % 2]

        @pl.when(t < _NTASK)
        def _(ti=ti, t=t, stag=stag, osem=osem):
            f = t >> 2
            row8 = t * 8

            def start(c, buf, sem, w=_CW):
                return pltpu.async_copy(
                    tab_hbm.at[pl.ds(row8, 8), pl.ds(c * _CW, w)],
                    buf.at[:, pl.ds(0, w)] if w != _CW else buf,
                    sem,
                )

            def wait(c, buf, sem, w=_CW):
                pltpu.make_async_copy(
                    tab_hbm.at[pl.ds(row8, 8), pl.ds(c * _CW, w)],
                    buf.at[:, pl.ds(0, w)] if w != _CW else buf,
                    sem,
                ).wait()

            start(0, buf0, sem0)
            start(1, buf1, sem1)
            pltpu.sync_copy(x_hbm.at[pl.ds(f * 32, 32), :], xv)
            pltpu.sync_copy(tail_hbm.at[pl.ds(f * 8, 8), :], tailv)
            if ti >= 2:
                out_dma(ti - 2, stag, osem).wait()

            def h0(c, carry):
                plsc.store_scatter(hist, [splat(c), lane], zeros)
                return carry

            lax.fori_loop(0, _NCH, h0, 0)

            def h1(i, carry):
                col = (i & 7) * 16 + lane
                v = plsc.load_gather(xv, [splat(i >> 3), col])
                plsc.addupdate_scatter(hist, [v >> 11, lane], ones)
                return carry

            lax.fori_loop(0, 256, h1, 0)

            def h2(c, running):
                row = plsc.load_gather(hist, [splat(c), lane])
                ex = plsc.cumsum(row) - row
                plsc.store_scatter(cur, [splat(c), lane], splat(running) + ex)
                cs[c] = running
                return running + jnp.sum(row)

            lax.fori_loop(0, _NCH, h2, jnp.int32(0))
            cs[_NCH] = BATCH

            def h3(i, carry):
                col = (i & 7) * 16 + lane
                v = plsc.load_gather(xv, [splat(i >> 3), col])
                b = (i >> 3) * 128 + col
                c = v >> 11
                pos = plsc.load_gather(cur, [c, lane])
                plsc.store_scatter(pv, [pos], v)
                plsc.store_scatter(pb, [pos], b)
                plsc.addupdate_scatter(cur, [c, lane], ones)
                return carry

            lax.fori_loop(0, 256, h3, 0)

            def process(c, buf, last):
                lo = cs[c]
                hi = cs[c + 1]

                def g(gi, carry):
                    pos = gi * 16 + lane
                    m = (pos >= lo) & (pos < hi)
                    v = plsc.load_gather(pv, [pos], mask=m)
                    b = plsc.load_gather(pb, [pos], mask=m)
                    vloc = jnp.maximum(v - c * _CW, 0)
                    if last:
                        mt = v >= _TAIL0
                        vt = jnp.maximum(v - _TAIL0, 0)
                    for k in range(8):
                        val = plsc.load_gather(buf, [splat(k), vloc], mask=m)
                        if last:
                            tval = plsc.load_gather(
                                tailv,
                                [splat(2 * (t & 3) + k // 4),
                                 (k % 4) * 32 + vt],
                                mask=m,
                            )
                            val = jnp.where(mt, tval, val)
                        plsc.store_scatter(stag, [splat(k), b], val, mask=m)
                    return carry

                lax.fori_loop(lo >> 4, (hi + 15) >> 4, g, 0)

            def chunk_pair(c2, carry):
                c = 2 * c2
                wait(c, buf0, sem0)
                process(c, buf0, False)

                @pl.when(c2 < _NFULL // 2 - 1)
                def _():
                    start(c + 2, buf0, sem0)

                @pl.when(c2 == _NFULL // 2 - 1)
                def _():
                    start(_NFULL, buf0, sem0, _LASTW)

                wait(c + 1, buf1, sem1)
                process(c + 1, buf1, False)

                @pl.when(c2 < _NFULL // 2 - 1)
                def _():
                    start(c + 3, buf1, sem1)

                return carry

            lax.fori_loop(0, _NFULL // 2, chunk_pair, 0)
            wait(_NFULL, buf0, sem0, _LASTW)
            process(_NFULL, buf0, True)

            out_dma(ti, stag, osem).start()

    @pl.when(wid >= 8)
    def _():
        out_dma(1, stags[1], osems[1]).wait()

    out_dma(2, stags[0], osems[0]).wait()

    @pl.when(wid + _NW * 3 < _NTASK)
    def _():
        out_dma(3, stags[1], osems[1]).wait()


_gather = pl.kernel(
    _body,
    out_type=jax.ShapeDtypeStruct((_NROWS, BATCH), jnp.float32),
    mesh=plsc.VectorSubcoreMesh(core_axis_name="c", subcore_axis_name="s"),
    scratch_types=[
        pltpu.VMEM((32, 128), jnp.int32),
        pltpu.VMEM((8, _CW), jnp.float32),
        pltpu.VMEM((8, _CW), jnp.float32),
        pltpu.VMEM((8, BATCH), jnp.float32),
        pltpu.VMEM((8, BATCH), jnp.float32),
        pltpu.VMEM((BATCH,), jnp.int32),
        pltpu.VMEM((BATCH,), jnp.int32),
        pltpu.VMEM((_NCH, 16), jnp.int32),
        pltpu.VMEM((_NCH, 16), jnp.int32),
        pltpu.VMEM((8, 128), jnp.float32),
        pltpu.SMEM((_NCH + 1,), jnp.int32),
        pltpu.SemaphoreType.DMA,
        pltpu.SemaphoreType.DMA,
        pltpu.SemaphoreType.DMA,
        pltpu.SemaphoreType.DMA,
    ],
    compiler_params=pltpu.CompilerParams(
        use_tc_tiling_on_sc=True, needs_layout_passes=False
    ),
)


@jax.jit
def kernel(x, tables):
    xi = x.astype(jnp.int32)
    x2 = xi.reshape(_NROWS, 128)
    tt2 = jnp.transpose(tables, (0, 2, 1)).reshape(_NROWS, VOCAB)
    tail2 = jnp.transpose(tables[:, _TAIL0:, :], (0, 2, 1)).reshape(208, 128)
    out = _gather(x2, tt2, tail2)
    return out.T


# speedup vs baseline: 5.3603x; 1.1022x over previous
import jax
import jax.numpy as jnp
from jax import lax
from jax.experimental import pallas as pl
from jax.experimental.pallas import tpu as pltpu
from jax.experimental.pallas import tpu_sc as plsc

NUM_FIELDS = 26
VOCAB = 100000
EMBED_DIM = 32
BATCH = 4096

_NC = 2
_NS = 16
_NW = _NC * _NS
_NROWS = NUM_FIELDS * EMBED_DIM
_NTASK = _NROWS // 8
_CW = 4096
_NFULL = 24
_NCH = _NFULL + 1
_LASTW = 99968 - _NFULL * _CW
_TAIL0 = 99968


def _body(x_hbm, tab_hbm, tail_hbm, out_hbm,
          xv, buf0, buf1, stag, pv, pb, hist, cur, tailv, cs,
          sem0, sem1, osem):
    wid = lax.axis_index("s") * _NC + lax.axis_index("c")
    lane = lax.iota(jnp.int32, 16)
    ones = jnp.ones((16,), jnp.int32)
    zeros = jnp.zeros((16,), jnp.int32)

    def splat(s):
        return jnp.broadcast_to(jnp.int32(s) if isinstance(s, int) else s, (16,))

    def out_dma(t):
        return pltpu.make_async_copy(
            stag, out_hbm.at[pl.ds(t * 8, 8), :], osem
        )

    def task_body(ti, carry):
        t = wid + _NW * ti

        @pl.when(t < _NTASK)
        def _():
            f = t >> 2
            row8 = t * 8

            def start(c, buf, sem, w=_CW):
                return pltpu.async_copy(
                    tab_hbm.at[pl.ds(row8, 8), pl.ds(c * _CW, w)],
                    buf.at[:, pl.ds(0, w)] if w != _CW else buf,
                    sem,
                )

            def wait(c, buf, sem, w=_CW):
                pltpu.make_async_copy(
                    tab_hbm.at[pl.ds(row8, 8), pl.ds(c * _CW, w)],
                    buf.at[:, pl.ds(0, w)] if w != _CW else buf,
                    sem,
                ).wait()

            start(0, buf0, sem0)
            start(1, buf1, sem1)
            pltpu.sync_copy(x_hbm.at[pl.ds(f * 32, 32), :], xv)
            pltpu.sync_copy(tail_hbm.at[pl.ds(f * 8, 8), :], tailv)

            def h0(c, carry):
                plsc.store_scatter(hist, [splat(c), lane], zeros)
                return carry

            lax.fori_loop(0, _NCH, h0, 0)

            def h1(i, carry):
                col = (i & 7) * 16 + lane
                v = plsc.load_gather(xv, [splat(i >> 3), col])
                plsc.addupdate_scatter(hist, [v >> 12, lane], ones)
                return carry

            lax.fori_loop(0, 256, h1, 0)

            def h2(c, running):
                row = plsc.load_gather(hist, [splat(c), lane])
                ex = plsc.cumsum(row) - row
                plsc.store_scatter(cur, [splat(c), lane], splat(running) + ex)
                cs[c] = running
                return running + jnp.sum(row)

            lax.fori_loop(0, _NCH, h2, jnp.int32(0))
            cs[_NCH] = BATCH

            def h3(i, carry):
                col = (i & 7) * 16 + lane
                v = plsc.load_gather(xv, [splat(i >> 3), col])
                b = (i >> 3) * 128 + col
                c = v >> 12
                pos = plsc.load_gather(cur, [c, lane])
                plsc.store_scatter(pv, [pos], v)
                plsc.store_scatter(pb, [pos], b)
                plsc.addupdate_scatter(cur, [c, lane], ones)
                return carry

            lax.fori_loop(0, 256, h3, 0)

            @pl.when(ti > 0)
            def _():
                out_dma(t - _NW).wait()

            def process(c, buf, last):
                lo = cs[c]
                hi = cs[c + 1]

                def g(gi, carry):
                    pos = gi * 16 + lane
                    m = (pos >= lo) & (pos < hi)
                    v = plsc.load_gather(pv, [pos], mask=m)
                    b = plsc.load_gather(pb, [pos], mask=m)
                    vloc = jnp.maximum(v - c * _CW, 0)
                    if last:
                        mt = v >= _TAIL0
                        vt = jnp.maximum(v - _TAIL0, 0)
                    for k in range(8):
                        val = plsc.load_gather(buf, [splat(k), vloc], mask=m)
                        if last:
                            tval = plsc.load_gather(
                                tailv,
                                [splat(2 * (t & 3) + k // 4),
                                 (k % 4) * 32 + vt],
                                mask=m,
                            )
                            val = jnp.where(mt, tval, val)
                        plsc.store_scatter(stag, [splat(k), b], val, mask=m)
                    return carry

                lax.fori_loop(lo >> 4, (hi + 15) >> 4, g, 0)

            def chunk_pair(c2, carry):
                c = 2 * c2
                wait(c, buf0, sem0)
                process(c, buf0, False)

                @pl.when(c2 < _NFULL // 2 - 1)
                def _():
                    start(c + 2, buf0, sem0)

                @pl.when(c2 == _NFULL // 2 - 1)
                def _():
                    start(_NFULL, buf0, sem0, _LASTW)

                wait(c + 1, buf1, sem1)
                process(c + 1, buf1, False)

                @pl.when(c2 < _NFULL // 2 - 1)
                def _():
                    start(c + 3, buf1, sem1)

                return carry

            lax.fori_loop(0, _NFULL // 2, chunk_pair, 0)
            wait(_NFULL, buf0, sem0, _LASTW)
            process(_NFULL, buf0, True)

            out_dma(t).start()

        return carry

    lax.fori_loop(0, 4, task_body, 0)

    t_last = wid + _NW * jnp.where(wid < 8, 3, 2)
    out_dma(t_last).wait()


_gather = pl.kernel(
    _body,
    out_type=jax.ShapeDtypeStruct((_NROWS, BATCH), jnp.float32),
    mesh=plsc.VectorSubcoreMesh(core_axis_name="c", subcore_axis_name="s"),
    scratch_types=[
        pltpu.VMEM((32, 128), jnp.int32),
        pltpu.VMEM((8, _CW), jnp.float32),
        pltpu.VMEM((8, _CW), jnp.float32),
        pltpu.VMEM((8, BATCH), jnp.float32),
        pltpu.VMEM((BATCH,), jnp.int32),
        pltpu.VMEM((BATCH,), jnp.int32),
        pltpu.VMEM((_NCH, 16), jnp.int32),
        pltpu.VMEM((_NCH, 16), jnp.int32),
        pltpu.VMEM((8, 128), jnp.float32),
        pltpu.SMEM((_NCH + 1,), jnp.int32),
        pltpu.SemaphoreType.DMA,
        pltpu.SemaphoreType.DMA,
        pltpu.SemaphoreType.DMA,
    ],
    compiler_params=pltpu.CompilerParams(
        use_tc_tiling_on_sc=True, needs_layout_passes=False
    ),
)


@jax.jit
def kernel(x, tables):
    xi = x.astype(jnp.int32)
    x2 = xi.reshape(_NROWS, 128)
    tt2 = jnp.transpose(tables, (0, 2, 1)).reshape(_NROWS, VOCAB)
    tail2 = jnp.transpose(tables[:, _TAIL0:, :], (0, 2, 1)).reshape(208, 128)
    out = _gather(x2, tt2, tail2)
    return out.T


# speedup vs baseline: 5.6273x; 1.0498x over previous
import jax
import jax.numpy as jnp
from jax import lax
from jax.experimental import pallas as pl
from jax.experimental.pallas import tpu as pltpu
from jax.experimental.pallas import tpu_sc as plsc

NUM_FIELDS = 26
VOCAB = 100000
EMBED_DIM = 32
BATCH = 4096

_NC = 2
_NS = 16
_NW = _NC * _NS
_NROWS = NUM_FIELDS * EMBED_DIM
_NTASK = _NROWS // 8
_CW = 3072
_NFULL = 32
_NCH = _NFULL + 1
_LASTW = 99968 - _NFULL * _CW
_TAIL0 = 99968


def _bucket(v):
    return ((v >> 10) * 21846) >> 16


def _body(x_hbm, tab_hbm, tail_hbm, out_hbm,
          xv, buf0, buf1, buf2, stag, pv, pb, hist, cur, tailv, cs,
          sem0, sem1, sem2, osem):
    wid = lax.axis_index("s") * _NC + lax.axis_index("c")
    lane = lax.iota(jnp.int32, 16)
    ones = jnp.ones((16,), jnp.int32)
    zeros = jnp.zeros((16,), jnp.int32)
    bufs = (buf0, buf1, buf2)
    sems = (sem0, sem1, sem2)

    def splat(s):
        return jnp.broadcast_to(jnp.int32(s) if isinstance(s, int) else s, (16,))

    def out_dma(t):
        return pltpu.make_async_copy(
            stag, out_hbm.at[pl.ds(t * 8, 8), :], osem
        )

    def task_body(ti, carry):
        t = wid + _NW * ti

        @pl.when(t < _NTASK)
        def _():
            f = t >> 2
            row8 = t * 8

            def start(c, buf, sem, w=_CW):
                return pltpu.async_copy(
                    tab_hbm.at[pl.ds(row8, 8), pl.ds(c * _CW, w)],
                    buf.at[:, pl.ds(0, w)] if w != _CW else buf,
                    sem,
                )

            def wait(c, buf, sem, w=_CW):
                pltpu.make_async_copy(
                    tab_hbm.at[pl.ds(row8, 8), pl.ds(c * _CW, w)],
                    buf.at[:, pl.ds(0, w)] if w != _CW else buf,
                    sem,
                ).wait()

            start(0, buf0, sem0)
            start(1, buf1, sem1)
            start(2, buf2, sem2)
            pltpu.sync_copy(x_hbm.at[pl.ds(f * 32, 32), :], xv)
            pltpu.sync_copy(tail_hbm.at[pl.ds(f * 8, 8), :], tailv)

            def h0(c, carry):
                plsc.store_scatter(hist, [splat(c), lane], zeros)
                return carry

            lax.fori_loop(0, _NCH, h0, 0)

            def h1(i, carry):
                col = (i & 7) * 16 + lane
                v = plsc.load_gather(xv, [splat(i >> 3), col])
                plsc.addupdate_scatter(hist, [_bucket(v), lane], ones)
                return carry

            lax.fori_loop(0, 256, h1, 0, unroll=4)

            def h2(c, running):
                row = plsc.load_gather(hist, [splat(c), lane])
                ex = plsc.cumsum(row) - row
                plsc.store_scatter(cur, [splat(c), lane], splat(running) + ex)
                cs[c] = running
                return running + jnp.sum(row)

            lax.fori_loop(0, _NCH, h2, jnp.int32(0))
            cs[_NCH] = BATCH

            def h3(i, carry):
                col = (i & 7) * 16 + lane
                v = plsc.load_gather(xv, [splat(i >> 3), col])
                b = (i >> 3) * 128 + col
                c = _bucket(v)
                pos = plsc.load_gather(cur, [c, lane])
                plsc.store_scatter(pv, [pos], v)
                plsc.store_scatter(pb, [pos], b)
                plsc.addupdate_scatter(cur, [c, lane], ones)
                return carry

            lax.fori_loop(0, 256, h3, 0, unroll=4)

            @pl.when(ti > 0)
            def _():
                out_dma(t - _NW).wait()

            def process(c, buf, last):
                lo = cs[c]
                hi = cs[c + 1]

                def g(gi, carry):
                    pos = gi * 16 + lane
                    m = (pos >= lo) & (pos < hi)
                    v = plsc.load_gather(pv, [pos], mask=m)
                    b = plsc.load_gather(pb, [pos], mask=m)
                    vloc = jnp.maximum(v - c * _CW, 0)
                    if last:
                        mt = v >= _TAIL0
                        vt = jnp.maximum(v - _TAIL0, 0)
                    for k in range(8):
                        val = plsc.load_gather(buf, [splat(k), vloc], mask=m)
                        if last:
                            tval = plsc.load_gather(
                                tailv,
                                [splat(2 * (t & 3) + k // 4),
                                 (k % 4) * 32 + vt],
                                mask=m,
                            )
                            val = jnp.where(mt, tval, val)
                        plsc.store_scatter(stag, [splat(k), b], val, mask=m)
                    return carry

                lax.fori_loop(lo >> 4, (hi + 15) >> 4, g, 0)

            def chunk_trip(c3, carry):
                c0 = 3 * c3
                for p in range(3):
                    c = c0 + p
                    wait(c, bufs[p], sems[p])
                    process(c, bufs[p], False)

                    @pl.when(c + 3 < _NFULL)
                    def _(p=p, c=c):
                        start(c + 3, bufs[p], sems[p])

                    @pl.when(c + 3 == _NFULL)
                    def _(p=p, c=c):
                        start(_NFULL, bufs[p], sems[p], _LASTW)

                return carry

            lax.fori_loop(0, _NFULL // 3, chunk_trip, 0)
            for p, c in ((0, _NFULL - 2), (1, _NFULL - 1)):
                wait(c, bufs[p], sems[p])
                process(c, bufs[p], False)
            wait(_NFULL, bufs[2], sems[2], _LASTW)
            process(_NFULL, bufs[2], True)

            out_dma(t).start()

        return carry

    lax.fori_loop(0, 4, task_body, 0)

    t_last = wid + _NW * jnp.where(wid < 8, 3, 2)
    out_dma(t_last).wait()


_gather = pl.kernel(
    _body,
    out_type=jax.ShapeDtypeStruct((_NROWS, BATCH), jnp.float32),
    mesh=plsc.VectorSubcoreMesh(core_axis_name="c", subcore_axis_name="s"),
    scratch_types=[
        pltpu.VMEM((32, 128), jnp.int32),
        pltpu.VMEM((8, _CW), jnp.float32),
        pltpu.VMEM((8, _CW), jnp.float32),
        pltpu.VMEM((8, _CW), jnp.float32),
        pltpu.VMEM((8, BATCH), jnp.float32),
        pltpu.VMEM((BATCH,), jnp.int32),
        pltpu.VMEM((BATCH,), jnp.int32),
        pltpu.VMEM((_NCH, 16), jnp.int32),
        pltpu.VMEM((_NCH, 16), jnp.int32),
        pltpu.VMEM((8, 128), jnp.float32),
        pltpu.SMEM((_NCH + 1,), jnp.int32),
        pltpu.SemaphoreType.DMA,
        pltpu.SemaphoreType.DMA,
        pltpu.SemaphoreType.DMA,
        pltpu.SemaphoreType.DMA,
    ],
    compiler_params=pltpu.CompilerParams(
        use_tc_tiling_on_sc=True, needs_layout_passes=False
    ),
)


@jax.jit
def kernel(x, tables):
    xi = x.astype(jnp.int32)
    x2 = xi.reshape(_NROWS, 128)
    tt2 = jnp.transpose(tables, (0, 2, 1)).reshape(_NROWS, VOCAB)
    tail2 = jnp.transpose(tables[:, _TAIL0:, :], (0, 2, 1)).reshape(208, 128)
    out = _gather(x2, tt2, tail2)
    return out.T
